# baseline (speedup 1.0000x reference)
import functools

import jax
import jax.numpy as jnp
from jax.experimental import pallas as pl
from jax.experimental.pallas import tpu as pltpu

_VMEM_LIMIT = 64 * 1024 * 1024


def _shift_lanes(v, k):
    if k > 0:
        return jnp.concatenate([v[:, k:], jnp.zeros((v.shape[0], k), v.dtype)],
                               axis=1)
    if k < 0:
        return jnp.concatenate([jnp.zeros((v.shape[0], -k), v.dtype),
                                v[:, :k]], axis=1)
    return v


def _taps(x2, W, mask_l, mask_r):
    pieces = []
    for kh in range(3):
        for kw in range(3):
            p = _shift_lanes(x2, (kh - 1) * W + (kw - 1))
            if kw == 0:
                p = p * mask_l
            elif kw == 2:
                p = p * mask_r
            pieces.append(p)
    return jnp.concatenate(pieces, axis=0)


def _conv_kernel(xt_ref, xm_ref, xb_ref, w_ref, wt_ref, wb_ref,
                 y_ref, s_ref, q_ref, *, H, W, DB, NB):
    dblk = pl.program_id(1)
    Cin = xt_ref.shape[1]
    HW = H * W
    Cout = wt_ref.shape[0]

    col = jax.lax.broadcasted_iota(jnp.int32, (1, HW), 1) % W
    mask_l = jnp.where(col == 0, 0.0, 1.0).astype(jnp.bfloat16)
    mask_r = jnp.where(col == W - 1, 0.0, 1.0).astype(jnp.bfloat16)
    top = jnp.where(dblk > 0, 1.0, 0.0)
    bot = jnp.where(dblk < NB - 1, 1.0, 0.0)

    xall = xm_ref[0].reshape(Cin * DB, HW).astype(jnp.bfloat16)
    pt = _taps(xall, W, mask_l, mask_r)
    ones = jnp.ones((8, HW), jnp.bfloat16)
    pt = jnp.concatenate([pt, ones], axis=0)
    acc = jnp.dot(w_ref[...], pt,
                  preferred_element_type=jnp.float32)

    et = (xt_ref[0, :, DB - 1, :] * top).astype(jnp.bfloat16)
    eb = (xb_ref[0, :, 0, :] * bot).astype(jnp.bfloat16)
    eacc_t = jnp.dot(wt_ref[...], _taps(et, W, mask_l, mask_r),
                     preferred_element_type=jnp.float32)
    eacc_b = jnp.dot(wb_ref[...], _taps(eb, W, mask_l, mask_r),
                     preferred_element_type=jnp.float32)

    for di in range(DB):
        a = acc[di * Cout:(di + 1) * Cout]
        if di == 0:
            a = a + eacc_t
        if di == DB - 1:
            a = a + eacc_b
        y_ref[0, :, di] = a.astype(y_ref.dtype)
        s_ref[0, di] = jnp.sum(a, axis=1, keepdims=True)
        q_ref[0, di] = jnp.sum(a * a, axis=1, keepdims=True)


def _bn_relu_kernel(y_ref, s_ref, q_ref, g_ref, be_ref, o_ref, *, count):
    mean = jnp.sum(s_ref[...], axis=0) / count
    var = jnp.maximum(jnp.sum(q_ref[...], axis=0) / count - mean * mean, 0.0)
    inv = g_ref[...] / jnp.sqrt(var + 1e-5)
    shift = be_ref[...] - mean * inv
    sc = inv.reshape(-1, 1, 1)
    sh = shift.reshape(-1, 1, 1)
    z = y_ref[0].astype(jnp.float32) * sc + sh
    o_ref[0] = jnp.maximum(z, 0.0)


def kernel(x, w, b, gamma, beta, alpha):
    N, Cin, D, H, W = x.shape
    Cout = w.shape[0]
    HW = H * W
    DB = 8
    while D % DB != 0:
        DB //= 2
    NB = D // DB
    K = 9 * Cin * DB

    x4 = x.reshape(N, Cin, D, HW)
    wt = jnp.transpose(w, (0, 3, 4, 1, 2)).reshape(Cout, 9, Cin, 3)
    ar = jnp.arange(DB)
    eye = (ar[:, None, None] + jnp.arange(3)[None, None, :] - 1
           == ar[None, :, None]).astype(w.dtype)
    w_all = jnp.einsum('ctik,dek->dctie', wt, eye)
    w_all = w_all.reshape(DB * Cout, K)
    bias_cols = jnp.zeros((DB * Cout, 8), w.dtype)
    bias_cols = bias_cols.at[:, 0].set(jnp.broadcast_to(b, (DB, Cout)).reshape(-1))
    w_all = jnp.concatenate([w_all, bias_cols], axis=1).astype(jnp.bfloat16)
    w_top = wt[:, :, :, 0].reshape(Cout, 9 * Cin).astype(jnp.bfloat16)
    w_bot = wt[:, :, :, 2].reshape(Cout, 9 * Cin).astype(jnp.bfloat16)

    def x_spec(shift):
        return pl.BlockSpec(
            (1, Cin, DB, HW),
            lambda n, i, s=shift: (n, 0, jnp.clip(i + s, 0, NB - 1), 0))

    y, s_sum, s_sq = pl.pallas_call(
        functools.partial(_conv_kernel, H=H, W=W, DB=DB, NB=NB),
        out_shape=(
            jax.ShapeDtypeStruct((N, Cout, D, HW), jnp.bfloat16),
            jax.ShapeDtypeStruct((N, D, Cout, 1), jnp.float32),
            jax.ShapeDtypeStruct((N, D, Cout, 1), jnp.float32),
        ),
        grid_spec=pltpu.PrefetchScalarGridSpec(
            num_scalar_prefetch=0,
            grid=(N, NB),
            in_specs=[
                x_spec(-1), x_spec(0), x_spec(1),
                pl.BlockSpec((DB * Cout, K + 8), lambda n, i: (0, 0)),
                pl.BlockSpec((Cout, 9 * Cin), lambda n, i: (0, 0)),
                pl.BlockSpec((Cout, 9 * Cin), lambda n, i: (0, 0)),
            ],
            out_specs=(
                pl.BlockSpec((1, Cout, DB, HW), lambda n, i: (n, 0, i, 0)),
                pl.BlockSpec((1, DB, Cout, 1), lambda n, i: (n, i, 0, 0)),
                pl.BlockSpec((1, DB, Cout, 1), lambda n, i: (n, i, 0, 0)),
            ),
        ),
        compiler_params=pltpu.CompilerParams(
            dimension_semantics=("parallel", "parallel"),
            vmem_limit_bytes=_VMEM_LIMIT,
        ),
    )(x4, x4, x4, w_all, w_top, w_bot)

    out = pl.pallas_call(
        functools.partial(_bn_relu_kernel, count=float(N * D * HW)),
        out_shape=jax.ShapeDtypeStruct((N, Cout, D, HW), jnp.float32),
        grid_spec=pltpu.PrefetchScalarGridSpec(
            num_scalar_prefetch=0,
            grid=(N, NB),
            in_specs=[
                pl.BlockSpec((1, Cout, DB, HW), lambda n, i: (n, 0, i, 0)),
                pl.BlockSpec((N * D, Cout, 1), lambda n, i: (0, 0, 0)),
                pl.BlockSpec((N * D, Cout, 1), lambda n, i: (0, 0, 0)),
                pl.BlockSpec((Cout, 1), lambda n, i: (0, 0)),
                pl.BlockSpec((Cout, 1), lambda n, i: (0, 0)),
            ],
            out_specs=pl.BlockSpec((1, Cout, DB, HW),
                                   lambda n, i: (n, 0, i, 0)),
        ),
        compiler_params=pltpu.CompilerParams(
            dimension_semantics=("parallel", "parallel"),
            vmem_limit_bytes=_VMEM_LIMIT,
        ),
    )(y, s_sum.reshape(N * D, Cout, 1), s_sq.reshape(N * D, Cout, 1),
      gamma.reshape(Cout, 1).astype(jnp.float32),
      beta.reshape(Cout, 1).astype(jnp.float32))
    return out.reshape(N, Cout, D, H, W)


# speedup vs baseline: 1.3788x; 1.3788x over previous
import functools

import jax
import jax.numpy as jnp
from jax.experimental import pallas as pl
from jax.experimental.pallas import tpu as pltpu

_VMEM_LIMIT = 64 * 1024 * 1024


def _shift_lanes(v, k):
    if k > 0:
        return jnp.concatenate([v[:, k:], jnp.zeros((v.shape[0], k), v.dtype)],
                               axis=1)
    if k < 0:
        return jnp.concatenate([jnp.zeros((v.shape[0], -k), v.dtype),
                                v[:, :k]], axis=1)
    return v


def _taps(x2, W, mask_l, mask_r):
    pieces = []
    for kh in range(3):
        for kw in range(3):
            p = _shift_lanes(x2, (kh - 1) * W + (kw - 1))
            if kw == 0:
                p = p * mask_l
            elif kw == 2:
                p = p * mask_r
            pieces.append(p)
    return jnp.concatenate(pieces, axis=0)


def _conv_kernel(xt_ref, xm_ref, xb_ref, w_ref, wt_ref, wb_ref,
                 y_ref, s_ref, q_ref, *, H, W, DB, NB):
    dblk = pl.program_id(1)
    Cin = xt_ref.shape[1]
    HW = H * W
    Cout = wt_ref.shape[0]

    col = jax.lax.broadcasted_iota(jnp.int32, (1, HW), 1) % W
    mask_l = jnp.where(col == 0, 0.0, 1.0).astype(jnp.bfloat16)
    mask_r = jnp.where(col == W - 1, 0.0, 1.0).astype(jnp.bfloat16)
    top = jnp.where(dblk > 0, 1.0, 0.0)
    bot = jnp.where(dblk < NB - 1, 1.0, 0.0)

    xall = xm_ref[0].reshape(Cin * DB, HW).astype(jnp.bfloat16)
    pt = _taps(xall, W, mask_l, mask_r)
    ones = jnp.ones((8, HW), jnp.bfloat16)
    pt = jnp.concatenate([pt, ones], axis=0)
    acc = jnp.dot(w_ref[...], pt,
                  preferred_element_type=jnp.float32)

    et = (xt_ref[0, :, DB - 1, :] * top).astype(jnp.bfloat16)
    eb = (xb_ref[0, :, 0, :] * bot).astype(jnp.bfloat16)
    eacc_t = jnp.dot(wt_ref[...], _taps(et, W, mask_l, mask_r),
                     preferred_element_type=jnp.float32)
    eacc_b = jnp.dot(wb_ref[...], _taps(eb, W, mask_l, mask_r),
                     preferred_element_type=jnp.float32)

    for di in range(DB):
        a = acc[di * Cout:(di + 1) * Cout]
        if di == 0:
            a = a + eacc_t
        if di == DB - 1:
            a = a + eacc_b
        y_ref[0, :, di] = a.astype(y_ref.dtype)
        s_ref[0, di] = jnp.sum(a, axis=1, keepdims=True)
        q_ref[0, di] = jnp.sum(a * a, axis=1, keepdims=True)


def _bn_relu_kernel(y_ref, s_ref, q_ref, g_ref, be_ref, o_ref, *, count):
    mean = jnp.sum(s_ref[...], axis=0) / count
    var = jnp.maximum(jnp.sum(q_ref[...], axis=0) / count - mean * mean, 0.0)
    inv = g_ref[...] / jnp.sqrt(var + 1e-5)
    shift = be_ref[...] - mean * inv
    sc = inv.reshape(-1, 1, 1)
    sh = shift.reshape(-1, 1, 1)
    z = y_ref[0].astype(jnp.float32) * sc + sh
    z = jnp.maximum(z, 0.0)
    o_ref[0] = z.reshape(o_ref.shape[1:])


def kernel(x, w, b, gamma, beta, alpha):
    N, Cin, D, H, W = x.shape
    Cout = w.shape[0]
    HW = H * W
    DB = 8
    while D % DB != 0:
        DB //= 2
    NB = D // DB
    K = 9 * Cin * DB

    x4 = x.reshape(N, Cin, D, HW)
    wt = jnp.transpose(w, (0, 3, 4, 1, 2)).reshape(Cout, 9, Cin, 3)
    ar = jnp.arange(DB)
    eye = (ar[:, None, None] + jnp.arange(3)[None, None, :] - 1
           == ar[None, :, None]).astype(w.dtype)
    w_all = jnp.einsum('ctik,dek->dctie', wt, eye)
    w_all = w_all.reshape(DB * Cout, K)
    bias_cols = jnp.zeros((DB * Cout, 8), w.dtype)
    bias_cols = bias_cols.at[:, 0].set(jnp.broadcast_to(b, (DB, Cout)).reshape(-1))
    w_all = jnp.concatenate([w_all, bias_cols], axis=1).astype(jnp.bfloat16)
    w_top = wt[:, :, :, 0].reshape(Cout, 9 * Cin).astype(jnp.bfloat16)
    w_bot = wt[:, :, :, 2].reshape(Cout, 9 * Cin).astype(jnp.bfloat16)

    def x_spec(shift):
        return pl.BlockSpec(
            (1, Cin, DB, HW),
            lambda n, i, s=shift: (n, 0, jnp.clip(i + s, 0, NB - 1), 0))

    y, s_sum, s_sq = pl.pallas_call(
        functools.partial(_conv_kernel, H=H, W=W, DB=DB, NB=NB),
        out_shape=(
            jax.ShapeDtypeStruct((N, Cout, D, HW), jnp.bfloat16),
            jax.ShapeDtypeStruct((N, D, Cout, 1), jnp.float32),
            jax.ShapeDtypeStruct((N, D, Cout, 1), jnp.float32),
        ),
        grid_spec=pltpu.PrefetchScalarGridSpec(
            num_scalar_prefetch=0,
            grid=(N, NB),
            in_specs=[
                x_spec(-1), x_spec(0), x_spec(1),
                pl.BlockSpec((DB * Cout, K + 8), lambda n, i: (0, 0)),
                pl.BlockSpec((Cout, 9 * Cin), lambda n, i: (0, 0)),
                pl.BlockSpec((Cout, 9 * Cin), lambda n, i: (0, 0)),
            ],
            out_specs=(
                pl.BlockSpec((1, Cout, DB, HW), lambda n, i: (n, 0, i, 0)),
                pl.BlockSpec((1, DB, Cout, 1), lambda n, i: (n, i, 0, 0)),
                pl.BlockSpec((1, DB, Cout, 1), lambda n, i: (n, i, 0, 0)),
            ),
        ),
        compiler_params=pltpu.CompilerParams(
            dimension_semantics=("parallel", "parallel"),
            vmem_limit_bytes=_VMEM_LIMIT,
        ),
    )(x4, x4, x4, w_all, w_top, w_bot)

    out = pl.pallas_call(
        functools.partial(_bn_relu_kernel, count=float(N * D * HW)),
        out_shape=jax.ShapeDtypeStruct((N, Cout, D, H, W), jnp.float32),
        grid_spec=pltpu.PrefetchScalarGridSpec(
            num_scalar_prefetch=0,
            grid=(N, NB),
            in_specs=[
                pl.BlockSpec((1, Cout, DB, HW), lambda n, i: (n, 0, i, 0)),
                pl.BlockSpec((N * D, Cout, 1), lambda n, i: (0, 0, 0)),
                pl.BlockSpec((N * D, Cout, 1), lambda n, i: (0, 0, 0)),
                pl.BlockSpec((Cout, 1), lambda n, i: (0, 0)),
                pl.BlockSpec((Cout, 1), lambda n, i: (0, 0)),
            ],
            out_specs=pl.BlockSpec((1, Cout, DB, H, W),
                                   lambda n, i: (n, 0, i, 0, 0)),
        ),
        compiler_params=pltpu.CompilerParams(
            dimension_semantics=("parallel", "parallel"),
            vmem_limit_bytes=_VMEM_LIMIT,
        ),
    )(y, s_sum.reshape(N * D, Cout, 1), s_sq.reshape(N * D, Cout, 1),
      gamma.reshape(Cout, 1).astype(jnp.float32),
      beta.reshape(Cout, 1).astype(jnp.float32))
    return out


# speedup vs baseline: 1.4046x; 1.0187x over previous
import functools

import jax
import jax.numpy as jnp
from jax.experimental import pallas as pl
from jax.experimental.pallas import tpu as pltpu

_VMEM_LIMIT = 64 * 1024 * 1024


def _shift_lanes(v, k):
    if k > 0:
        return jnp.concatenate([v[:, k:], jnp.zeros((v.shape[0], k), v.dtype)],
                               axis=1)
    if k < 0:
        return jnp.concatenate([jnp.zeros((v.shape[0], -k), v.dtype),
                                v[:, :k]], axis=1)
    return v


def _taps(x2, W, mask_l, mask_r):
    pieces = []
    for kh in range(3):
        for kw in range(3):
            p = _shift_lanes(x2, (kh - 1) * W + (kw - 1))
            if kw == 0:
                p = p * mask_l
            elif kw == 2:
                p = p * mask_r
            pieces.append(p)
    return jnp.concatenate(pieces, axis=0)


def _conv_kernel(xt_ref, xm_ref, xb_ref, w_ref,
                 y_ref, s_ref, q_ref, *, H, W, DB, NB, Cout):
    dblk = pl.program_id(1)
    Cin = xt_ref.shape[1]
    HW = H * W

    col = jax.lax.broadcasted_iota(jnp.int32, (1, HW), 1) % W
    mask_l = jnp.where(col == 0, 0.0, 1.0).astype(jnp.bfloat16)
    mask_r = jnp.where(col == W - 1, 0.0, 1.0).astype(jnp.bfloat16)
    top = jnp.where(dblk > 0, 1.0, 0.0)
    bot = jnp.where(dblk < NB - 1, 1.0, 0.0)

    xall = xm_ref[0].reshape(Cin * DB, HW).astype(jnp.bfloat16)
    et = (xt_ref[0, :, xt_ref.shape[2] - 1, :] * top).astype(jnp.bfloat16)
    eb = (xb_ref[0, :, 0, :] * bot).astype(jnp.bfloat16)
    pt = jnp.concatenate([
        _taps(xall, W, mask_l, mask_r),
        _taps(et, W, mask_l, mask_r),
        _taps(eb, W, mask_l, mask_r),
        jnp.ones((8, HW), jnp.bfloat16),
    ], axis=0)
    acc = jnp.dot(w_ref[...], pt,
                  preferred_element_type=jnp.float32)

    for di in range(DB):
        a = acc[di * Cout:(di + 1) * Cout]
        y_ref[0, :, di] = a.astype(y_ref.dtype)
        s_ref[0, di] = jnp.sum(a, axis=1, keepdims=True)
        q_ref[0, di] = jnp.sum(a * a, axis=1, keepdims=True)


def _bn_relu_kernel(y_ref, s_ref, q_ref, g_ref, be_ref, o_ref, *, count):
    mean = jnp.sum(s_ref[...], axis=0) / count
    var = jnp.maximum(jnp.sum(q_ref[...], axis=0) / count - mean * mean, 0.0)
    inv = g_ref[...] / jnp.sqrt(var + 1e-5)
    shift = be_ref[...] - mean * inv
    sc = inv.reshape(-1, 1, 1)
    sh = shift.reshape(-1, 1, 1)
    z = y_ref[0].astype(jnp.float32) * sc + sh
    z = jnp.maximum(z, 0.0)
    o_ref[0] = z.reshape(o_ref.shape[1:])


def kernel(x, w, b, gamma, beta, alpha):
    N, Cin, D, H, W = x.shape
    Cout = w.shape[0]
    HW = H * W
    DB = 8
    while D % DB != 0:
        DB //= 2
    NB = D // DB
    K = 9 * Cin * DB

    x4 = x.reshape(N, Cin, D, HW)
    wt = jnp.transpose(w, (0, 3, 4, 1, 2)).reshape(Cout, 9, Cin, 3)
    ar = jnp.arange(DB)
    eye = (ar[:, None, None] + jnp.arange(3)[None, None, :] - 1
           == ar[None, :, None]).astype(w.dtype)
    w_all = jnp.einsum('ctik,dek->dctie', wt, eye)
    w_all = w_all.reshape(DB * Cout, K)
    w_top = jnp.pad(wt[:, :, :, 0].reshape(Cout, 9 * Cin),
                    ((0, (DB - 1) * Cout), (0, 0)))
    w_bot = jnp.pad(wt[:, :, :, 2].reshape(Cout, 9 * Cin),
                    (((DB - 1) * Cout, 0), (0, 0)))
    bias_cols = jnp.pad(
        jnp.broadcast_to(b.astype(w.dtype), (DB, Cout)).reshape(DB * Cout, 1),
        ((0, 0), (0, 7)))
    w_all = jnp.concatenate([w_all, w_top, w_bot, bias_cols],
                            axis=1).astype(jnp.bfloat16)
    K2 = K + 18 * Cin + 8

    def h_spec(off):
        return pl.BlockSpec(
            (1, Cin, DB, HW),
            lambda n, i, o=off: (n, 0, jnp.clip(i + o, 0, NB - 1), 0))
    xt_spec, xb_spec = h_spec(-1), h_spec(1)
    xm_spec = pl.BlockSpec((1, Cin, DB, HW), lambda n, i: (n, 0, i, 0))

    y, s_sum, s_sq = pl.pallas_call(
        functools.partial(_conv_kernel, H=H, W=W, DB=DB, NB=NB, Cout=Cout),
        out_shape=(
            jax.ShapeDtypeStruct((N, Cout, D, HW), jnp.bfloat16),
            jax.ShapeDtypeStruct((N, D, Cout, 1), jnp.float32),
            jax.ShapeDtypeStruct((N, D, Cout, 1), jnp.float32),
        ),
        grid_spec=pltpu.PrefetchScalarGridSpec(
            num_scalar_prefetch=0,
            grid=(N, NB),
            in_specs=[
                xt_spec, xm_spec, xb_spec,
                pl.BlockSpec((DB * Cout, K2), lambda n, i: (0, 0)),
            ],
            out_specs=(
                pl.BlockSpec((1, Cout, DB, HW), lambda n, i: (n, 0, i, 0)),
                pl.BlockSpec((1, DB, Cout, 1), lambda n, i: (n, i, 0, 0)),
                pl.BlockSpec((1, DB, Cout, 1), lambda n, i: (n, i, 0, 0)),
            ),
        ),
        compiler_params=pltpu.CompilerParams(
            dimension_semantics=("parallel", "parallel"),
            vmem_limit_bytes=_VMEM_LIMIT,
        ),
    )(x4, x4, x4, w_all)

    out = pl.pallas_call(
        functools.partial(_bn_relu_kernel, count=float(N * D * HW)),
        out_shape=jax.ShapeDtypeStruct((N, Cout, D, H, W), jnp.float32),
        grid_spec=pltpu.PrefetchScalarGridSpec(
            num_scalar_prefetch=0,
            grid=(N, NB),
            in_specs=[
                pl.BlockSpec((1, Cout, DB, HW), lambda n, i: (n, 0, i, 0)),
                pl.BlockSpec((N * D, Cout, 1), lambda n, i: (0, 0, 0)),
                pl.BlockSpec((N * D, Cout, 1), lambda n, i: (0, 0, 0)),
                pl.BlockSpec((Cout, 1), lambda n, i: (0, 0)),
                pl.BlockSpec((Cout, 1), lambda n, i: (0, 0)),
            ],
            out_specs=pl.BlockSpec((1, Cout, DB, H, W),
                                   lambda n, i: (n, 0, i, 0, 0)),
        ),
        compiler_params=pltpu.CompilerParams(
            dimension_semantics=("parallel", "parallel"),
            vmem_limit_bytes=_VMEM_LIMIT,
        ),
    )(y, s_sum.reshape(N * D, Cout, 1), s_sq.reshape(N * D, Cout, 1),
      gamma.reshape(Cout, 1).astype(jnp.float32),
      beta.reshape(Cout, 1).astype(jnp.float32))
    return out


# speedup vs baseline: 1.4465x; 1.0298x over previous
import functools

import jax
import jax.numpy as jnp
from jax.experimental import pallas as pl
from jax.experimental.pallas import tpu as pltpu

_VMEM_LIMIT = 64 * 1024 * 1024


def _shift_lanes(v, k):
    if k > 0:
        return jnp.concatenate([v[:, k:], jnp.zeros((v.shape[0], k), v.dtype)],
                               axis=1)
    if k < 0:
        return jnp.concatenate([jnp.zeros((v.shape[0], -k), v.dtype),
                                v[:, :k]], axis=1)
    return v


def _taps(x2, W, mask_l, mask_r):
    pieces = []
    for kh in range(3):
        for kw in range(3):
            p = _shift_lanes(x2, (kh - 1) * W + (kw - 1))
            if kw == 0:
                p = p * mask_l
            elif kw == 2:
                p = p * mask_r
            pieces.append(p)
    return jnp.concatenate(pieces, axis=0)


def _conv_kernel(x_ref, w_ref, y_ref, s_ref, q_ref, *, H, W, DB, NB, Cout):
    Cin = x_ref.shape[1]
    HW = H * W

    col = jax.lax.broadcasted_iota(jnp.int32, (1, HW), 1) % W
    mask_l = jnp.where(col == 0, 0.0, 1.0).astype(jnp.bfloat16)
    mask_r = jnp.where(col == W - 1, 0.0, 1.0).astype(jnp.bfloat16)
    zrow = jnp.zeros((Cin, HW), jnp.bfloat16)

    for j in range(NB):
        xall = (x_ref[0, :, DB * j:DB * (j + 1), :]
                .reshape(Cin * DB, HW).astype(jnp.bfloat16))
        et = (x_ref[0, :, DB * j - 1, :].astype(jnp.bfloat16)
              if j > 0 else zrow)
        eb = (x_ref[0, :, DB * (j + 1), :].astype(jnp.bfloat16)
              if j < NB - 1 else zrow)
        pt = jnp.concatenate([
            _taps(xall, W, mask_l, mask_r),
            _taps(et, W, mask_l, mask_r),
            _taps(eb, W, mask_l, mask_r),
            jnp.ones((8, HW), jnp.bfloat16),
        ], axis=0)
        acc = jnp.dot(w_ref[...], pt,
                      preferred_element_type=jnp.float32)

        for di in range(DB):
            a = acc[di * Cout:(di + 1) * Cout]
            y_ref[0, :, DB * j + di] = a.astype(y_ref.dtype)
            s_ref[0, DB * j + di] = jnp.sum(a, axis=1, keepdims=True)
            q_ref[0, DB * j + di] = jnp.sum(a * a, axis=1, keepdims=True)


def _bn_relu_kernel(y_ref, s_ref, q_ref, g_ref, be_ref, o_ref, *, count):
    mean = jnp.sum(s_ref[...], axis=0) / count
    var = jnp.maximum(jnp.sum(q_ref[...], axis=0) / count - mean * mean, 0.0)
    inv = g_ref[...] / jnp.sqrt(var + 1e-5)
    shift = be_ref[...] - mean * inv
    sc = inv.reshape(-1, 1, 1)
    sh = shift.reshape(-1, 1, 1)
    z = y_ref[0].astype(jnp.float32) * sc + sh
    z = jnp.maximum(z, 0.0)
    o_ref[0] = z.reshape(o_ref.shape[1:])


def kernel(x, w, b, gamma, beta, alpha):
    N, Cin, D, H, W = x.shape
    Cout = w.shape[0]
    HW = H * W
    DB = 8
    while D % DB != 0:
        DB //= 2
    NB = D // DB
    K = 9 * Cin * DB

    x4 = x.reshape(N, Cin, D, HW)
    wt = jnp.transpose(w, (0, 3, 4, 1, 2)).reshape(Cout, 9, Cin, 3)
    ar = jnp.arange(DB)
    eye = (ar[:, None, None] + jnp.arange(3)[None, None, :] - 1
           == ar[None, :, None]).astype(w.dtype)
    w_all = jnp.einsum('ctik,dek->dctie', wt, eye)
    w_all = w_all.reshape(DB * Cout, K)
    w_top = jnp.pad(wt[:, :, :, 0].reshape(Cout, 9 * Cin),
                    ((0, (DB - 1) * Cout), (0, 0)))
    w_bot = jnp.pad(wt[:, :, :, 2].reshape(Cout, 9 * Cin),
                    (((DB - 1) * Cout, 0), (0, 0)))
    bias_cols = jnp.pad(
        jnp.broadcast_to(b.astype(w.dtype), (DB, Cout)).reshape(DB * Cout, 1),
        ((0, 0), (0, 7)))
    w_all = jnp.concatenate([w_all, w_top, w_bot, bias_cols],
                            axis=1).astype(jnp.bfloat16)
    K2 = K + 18 * Cin + 8

    xm_spec = pl.BlockSpec((1, Cin, D, HW), lambda n: (n, 0, 0, 0))

    y, s_sum, s_sq = pl.pallas_call(
        functools.partial(_conv_kernel, H=H, W=W, DB=DB, NB=NB, Cout=Cout),
        out_shape=(
            jax.ShapeDtypeStruct((N, Cout, D, HW), jnp.bfloat16),
            jax.ShapeDtypeStruct((N, D, Cout, 1), jnp.float32),
            jax.ShapeDtypeStruct((N, D, Cout, 1), jnp.float32),
        ),
        grid_spec=pltpu.PrefetchScalarGridSpec(
            num_scalar_prefetch=0,
            grid=(N,),
            in_specs=[
                xm_spec,
                pl.BlockSpec((DB * Cout, K2), lambda n: (0, 0)),
            ],
            out_specs=(
                pl.BlockSpec((1, Cout, D, HW), lambda n: (n, 0, 0, 0)),
                pl.BlockSpec((1, D, Cout, 1), lambda n: (n, 0, 0, 0)),
                pl.BlockSpec((1, D, Cout, 1), lambda n: (n, 0, 0, 0)),
            ),
        ),
        compiler_params=pltpu.CompilerParams(
            dimension_semantics=("parallel",),
            vmem_limit_bytes=_VMEM_LIMIT,
        ),
    )(x4, w_all)

    out = pl.pallas_call(
        functools.partial(_bn_relu_kernel, count=float(N * D * HW)),
        out_shape=jax.ShapeDtypeStruct((N, Cout, D, H, W), jnp.float32),
        grid_spec=pltpu.PrefetchScalarGridSpec(
            num_scalar_prefetch=0,
            grid=(N, NB),
            in_specs=[
                pl.BlockSpec((1, Cout, DB, HW), lambda n, i: (n, 0, i, 0)),
                pl.BlockSpec((N * D, Cout, 1), lambda n, i: (0, 0, 0)),
                pl.BlockSpec((N * D, Cout, 1), lambda n, i: (0, 0, 0)),
                pl.BlockSpec((Cout, 1), lambda n, i: (0, 0)),
                pl.BlockSpec((Cout, 1), lambda n, i: (0, 0)),
            ],
            out_specs=pl.BlockSpec((1, Cout, DB, H, W),
                                   lambda n, i: (n, 0, i, 0, 0)),
        ),
        compiler_params=pltpu.CompilerParams(
            dimension_semantics=("parallel", "parallel"),
            vmem_limit_bytes=_VMEM_LIMIT,
        ),
    )(y, s_sum.reshape(N * D, Cout, 1), s_sq.reshape(N * D, Cout, 1),
      gamma.reshape(Cout, 1).astype(jnp.float32),
      beta.reshape(Cout, 1).astype(jnp.float32))
    return out
